# Initial kernel scaffold; baseline (speedup 1.0000x reference)
import jax
import jax.numpy as jnp
from jax.experimental import pallas as pl


def kernel(predicted_locs, predicted_scores, boxes, labels, priors):
    raise NotImplementedError("write your pallas kernel here")


# speedup vs baseline: 3.8815x; 3.8815x over previous
import jax
import jax.numpy as jnp
from jax.experimental import pallas as pl

_SPLITS = (0, 6400, 8000, 8400, 8500, 8525)
_N_LEVELS = 5
_K = 9
_N_OBJ = 8
_N_CLASSES = 80
_BIG_F = 1e30


def _iota(shape, dim):
    return jax.lax.broadcasted_iota(jnp.int32, shape, dim)


def _lane_to_sublane(v, n):
    r = _iota((n, n), 0)
    c = _iota((n, n), 1)
    vb = jnp.broadcast_to(v, (n, n))
    zero = jnp.zeros((), v.dtype)
    return jnp.sum(jnp.where(r == c, vb, zero), axis=1, keepdims=True)


def _atan_nonneg(x):
    inv = x > 1.0
    y = jnp.where(inv, 1.0 / jnp.where(inv, x, 1.0), x)
    y = y / (1.0 + jnp.sqrt(1.0 + y * y))
    y = y / (1.0 + jnp.sqrt(1.0 + y * y))
    t = y * y
    s = y * (1.0 + t * (-1.0 / 3.0 + t * (1.0 / 5.0
                                          + t * (-1.0 / 7.0 + t / 9.0))))
    a = 4.0 * s
    return jnp.where(inv, jnp.pi / 2.0 - a, a)


def _softplus_pair(x):
    e = jnp.exp(-jnp.abs(x))
    l1pe = jnp.log1p(e)
    p = jnp.where(x >= 0.0, 1.0 / (1.0 + e), e / (1.0 + e))
    sp_pos = jnp.maximum(x, 0.0) + l1pe
    sp_neg = jnp.maximum(-x, 0.0) + l1pe
    return p, sp_pos, sp_neg


def _body(scores_ref, pos_ref, locs_ref, boxes_ref, labels_ref, priors_ref,
          out_ref):
    boxes = boxes_ref[0]
    bx1 = boxes[:, 0:1]
    by1 = boxes[:, 1:2]
    bx2 = boxes[:, 2:3]
    by2 = boxes[:, 3:4]
    gcx = (bx1 + bx2) * 0.5
    gcy = (by1 + by2) * 0.5
    area_a = (bx2 - bx1) * (by2 - by1)
    lab8 = _lane_to_sublane(labels_ref[0], _N_OBJ).astype(jnp.float32)

    px = priors_ref[0:1, :]
    py = priors_ref[1:2, :]
    pw = priors_ref[2:3, :]
    ph = priors_ref[3:4, :]
    locs = locs_ref[0]

    lvl = []
    for l in range(_N_LEVELS):
        s0, s1 = _SPLITS[l], _SPLITS[l + 1]
        npl = s1 - s0
        pxl = px[:, s0:s1]
        pyl = py[:, s0:s1]
        pwl = pw[:, s0:s1]
        phl = ph[:, s0:s1]
        plx1 = pxl - pwl / 2.0
        ply1 = pyl - phl / 2.0
        plx2 = pxl + pwl / 2.0
        ply2 = pyl + phl / 2.0
        dist = jnp.sqrt((gcx - pxl) ** 2 + (gcy - pyl) ** 2)
        inter = (jnp.clip(jnp.minimum(bx2, plx2) - jnp.maximum(bx1, plx1),
                          0.0, None)
                 * jnp.clip(jnp.minimum(by2, ply2) - jnp.maximum(by1, ply1),
                            0.0, None))
        area_b = (plx2 - plx1) * (ply2 - ply1)
        ov = inter / (area_a + area_b - inter + 1e-10)

        gxl = locs[0:1, s0:s1]
        gyl = locs[1:2, s0:s1]
        gwl = locs[2:3, s0:s1]
        ghl = locs[3:4, s0:s1]

        colio = _iota((_N_OBJ, npl), 1)
        gathered = [[] for _ in range(7)]
        for _j in range(_K):
            m = jnp.min(dist, axis=1, keepdims=True)
            idx = jnp.min(jnp.where(dist == m, colio, 2 ** 30),
                          axis=1, keepdims=True)
            hit = colio == idx

            def pick(x, hit=hit):
                xb = jnp.broadcast_to(x, hit.shape)
                return jnp.sum(jnp.where(hit, xb, 0.0), axis=1, keepdims=True)

            for dst, src in zip(gathered, (ov, pxl, pyl, gxl, gyl, gwl, ghl)):
                dst.append(pick(src))
            dist = jnp.where(hit, _BIG_F, dist)
        pov_l, pcx_l, pcy_l, ggx, ggy, ggw, ggh = (
            jnp.concatenate(g, axis=1) for g in gathered)
        lvl.append((pov_l, pcx_l, pcy_l, ggx, ggy, ggw, ggh))

    cat = jnp.concatenate([t[0] for t in lvl], axis=1)
    n_cand = cat.shape[1]
    mean = jnp.sum(cat, axis=1, keepdims=True) / n_cand
    var = jnp.sum((cat - mean) ** 2, axis=1, keepdims=True) / (n_cand - 1)
    thr = mean + jnp.sqrt(var)

    loc_num = jnp.zeros((1, 1), jnp.float32)
    sel_sum = jnp.zeros((1, 1), jnp.float32)
    lab45 = []
    rio = _iota((_N_OBJ, _K), 0)
    for l in range(_N_LEVELS):
        pov_l, pcx_l, pcy_l, ggx, ggy, ggw, ggh = lvl[l]
        inside = ((bx1 < pcx_l) & (pcx_l < bx2)
                  & (by1 < pcy_l) & (pcy_l < by2))
        mask = (pov_l > thr) & inside
        val = jnp.where(mask, pov_l, 0.0)
        bv = jnp.max(val, axis=0, keepdims=True)
        bo = jnp.min(jnp.where(val == bv, rio, _N_OBJ),
                     axis=0, keepdims=True)
        oh = rio == bo
        selp = (bv > 0.0).astype(jnp.float32)

        def rowpick(x, oh=oh):
            xb = jnp.broadcast_to(x, oh.shape)
            return jnp.sum(jnp.where(oh, xb, 0.0), axis=0, keepdims=True)

        lab45.append(selp * rowpick(lab8))
        tlx1 = rowpick(bx1)
        tly1 = rowpick(by1)
        tlx2 = rowpick(bx2)
        tly2 = rowpick(by2)
        gx = rowpick(ggx)
        gy = rowpick(ggy)
        gw = rowpick(ggw)
        gh = rowpick(ggh)
        pcx = rowpick(pcx_l)
        pcy = rowpick(pcy_l)

        s = (0.1, 0.2, 0.375, 0.55, 0.725)[l]
        dcx = gx * s / 10.0 + pcx
        dcy = gy * s / 10.0 + pcy
        dw = jnp.exp(gw / 5.0) * s
        dh = jnp.exp(gh / 5.0) * s
        dlx1 = dcx - dw / 2.0
        dly1 = dcy - dh / 2.0
        dlx2 = dcx + dw / 2.0
        dly2 = dcy + dh / 2.0

        eps = 1e-7
        pw_ = dlx2 - dlx1
        ph_ = dly2 - dly1
        tw_ = tlx2 - tlx1
        th_ = tly2 - tly1
        iw = jnp.clip(jnp.minimum(dlx2, tlx2) - jnp.maximum(dlx1, tlx1),
                      0.0, None)
        ih = jnp.clip(jnp.minimum(dly2, tly2) - jnp.maximum(dly1, tly1),
                      0.0, None)
        inter = iw * ih
        union = pw_ * ph_ + tw_ * th_ - inter + eps
        iou = inter / union
        cw = jnp.maximum(dlx2, tlx2) - jnp.minimum(dlx1, tlx1)
        ch = jnp.maximum(dly2, tly2) - jnp.minimum(dly1, tly1)
        c2 = cw ** 2 + ch ** 2 + eps
        rho2 = ((dlx1 + dlx2 - tlx1 - tlx2) ** 2
                + (dly1 + dly2 - tly1 - tly2) ** 2) / 4.0
        v = (4.0 / (jnp.pi ** 2)) * (_atan_nonneg(tw_ / (th_ + eps))
                                     - _atan_nonneg(pw_ / (ph_ + eps))) ** 2
        a = v / (1.0 - iou + v + eps)
        per = 1.0 - (iou - rho2 / c2 - a * v)

        loc_num = loc_num + jnp.sum(per * selp, axis=1, keepdims=True)
        sel_sum = sel_sum + jnp.sum(selp, axis=1, keepdims=True)

    lab45 = jnp.concatenate(lab45, axis=1)
    npos = jnp.sum((lab45 > 0.0).astype(jnp.float32), axis=1, keepdims=True)

    z = scores_ref[0]
    p, sp_pos, _ = _softplus_pair(z)
    bg = jnp.sum(0.75 * p * p * sp_pos)

    n_pad = pos_ref.shape[1]
    labp = jnp.concatenate(
        [lab45, jnp.zeros((1, n_pad - 45), jnp.float32)], axis=1)
    labc = _lane_to_sublane(labp, n_pad)
    zr = pos_ref[0]
    cio = _iota((n_pad, _N_CLASSES), 1).astype(jnp.float32)
    tmask = (cio == labc - 1.0) & (labc > 0.0)
    pr, spr_pos, spr_neg = _softplus_pair(zr)
    corr_terms = (0.25 * (1.0 - pr) ** 2 * spr_neg
                  - 0.75 * pr * pr * spr_pos)
    corr = jnp.sum(jnp.where(tmask, corr_terms, 0.0))

    focal = bg + corr

    oio = _iota((1, 128), 1)
    res = (jnp.where(oio == 0, focal, 0.0)
           + jnp.where(oio == 1, npos, 0.0)
           + jnp.where(oio == 2, loc_num, 0.0)
           + jnp.where(oio == 3, sel_sum, 0.0))
    out_ref[0] = res


def _impl(predicted_locs, predicted_scores, boxes, labels, priors,
          interpret=False):
    batch = predicted_locs.shape[0]
    n_pri = predicted_locs.shape[1]
    n_cls = predicted_scores.shape[2]

    locs_t = jnp.transpose(predicted_locs, (0, 2, 1))
    priors_t = jnp.transpose(priors, (1, 0))
    pos_rows = jnp.concatenate(
        [predicted_scores[:, s:s + _K, :] for s in _SPLITS[:-1]]
        + [jnp.zeros((batch, 3, n_cls), jnp.float32)], axis=1)

    parts = pl.pallas_call(
        _body,
        grid=(batch,),
        in_specs=[
            pl.BlockSpec((1, n_pri, n_cls), lambda i: (i, 0, 0)),
            pl.BlockSpec((1, 48, n_cls), lambda i: (i, 0, 0)),
            pl.BlockSpec((1, 4, n_pri), lambda i: (i, 0, 0)),
            pl.BlockSpec((1, _N_OBJ, 4), lambda i: (i, 0, 0)),
            pl.BlockSpec((1, 1, _N_OBJ), lambda i: (i, 0, 0)),
            pl.BlockSpec((4, n_pri), lambda i: (0, 0)),
        ],
        out_specs=pl.BlockSpec((1, 1, 128), lambda i: (i, 0, 0)),
        out_shape=jax.ShapeDtypeStruct((batch, 1, 128), jnp.float32),
        interpret=interpret,
    )(predicted_scores, pos_rows, locs_t, boxes,
      labels.reshape(batch, 1, _N_OBJ), priors_t)

    focal = jnp.sum(parts[:, 0, 0])
    npos = jnp.maximum(jnp.sum(parts[:, 0, 1]), 1.0)
    loc_num = jnp.sum(parts[:, 0, 2])
    sel_sum = jnp.maximum(jnp.sum(parts[:, 0, 3]), 1.0)
    return focal / npos + loc_num / sel_sum


def kernel(predicted_locs, predicted_scores, boxes, labels, priors):
    return _impl(predicted_locs, predicted_scores, boxes, labels, priors)
